# baseline (speedup 1.0000x reference)
import functools

import jax
import jax.numpy as jnp
from jax import lax
from jax.experimental import pallas as pl
from jax.experimental.pallas import tpu as pltpu
from jax.experimental.pallas import tpu_sc as plsc

N = 10000
E = 160000
D = 256
NC = 2
NS = 16
NW = NC * NS
L = 16

R = 320
NPAD = NW * R
CE = 2000
CB = 2048
GB = 64
NF = D // L


def _sc_segmax_body(x_hbm, src_hbm, dst_hbm, out_hbm,
                    acc, dstc, srcc, srcbuf, dlbuf, rows, sem):
    wid = lax.axis_index("s") * NC + lax.axis_index("c")
    lo = wid * R

    neg = jnp.full((L,), -jnp.inf, dtype=jnp.float32)
    zi = jnp.zeros((L,), dtype=jnp.int32)

    @pl.loop(0, R)
    def _(r):
        for f in range(NF):
            acc[r, pl.ds(f * L, L)] = neg

    @pl.loop(0, CB // L)
    def _(i):
        srcbuf[pl.ds(i * L, L)] = zi

    @pl.loop(0, E // CE)
    def _(c):
        pltpu.sync_copy(dst_hbm.at[pl.ds(c * CE, CE)], dstc)
        pltpu.sync_copy(src_hbm.at[pl.ds(c * CE, CE)], srcc)

        def filt(it, pos):
            dstv = dstc[pl.ds(it * L, L)]
            srcv = srcc[pl.ds(it * L, L)]
            mask = (dstv >= lo) & (dstv < lo + R)
            mi = jnp.where(mask, 1, 0).astype(jnp.int32)
            posv = pos + plsc.cumsum(mi) - mi
            plsc.store_scatter(srcbuf, [posv], srcv, mask=mask)
            plsc.store_scatter(dlbuf, [posv], dstv - lo, mask=mask)
            return pos + plsc.all_reduce_population_count(mask)

        pos = lax.fori_loop(0, CE // L, filt, jnp.zeros((L,), jnp.int32))
        m = jnp.max(pos)
        nb = (m + GB - 1) // GB

        def batch(b, _):
            pltpu.async_copy(
                x_hbm.at[srcbuf.at[pl.ds(b * GB, GB)]], rows, sem).wait()
            cnt = jnp.minimum(GB, m - b * GB)

            def upd(i, _):
                dl = dlbuf[pl.ds(b * GB + i, L)][0]
                for f in range(NF):
                    s = pl.ds(f * L, L)
                    acc[dl, s] = jnp.maximum(acc[dl, s], rows[i, s])
                return 0

            lax.fori_loop(0, cnt, upd, 0)
            return 0

        lax.fori_loop(0, nb, batch, 0)

    pltpu.sync_copy(acc, out_hbm.at[pl.ds(lo, R)])


@functools.cache
def _sc_segmax():
    return pl.kernel(
        _sc_segmax_body,
        out_type=jax.ShapeDtypeStruct((NPAD, D), jnp.float32),
        mesh=plsc.VectorSubcoreMesh(
            core_axis_name="c", subcore_axis_name="s",
            num_cores=NC, num_subcores=NS),
        compiler_params=pltpu.CompilerParams(needs_layout_passes=False),
        scratch_types=[
            pltpu.VMEM((R, D), jnp.float32),
            pltpu.VMEM((CE,), jnp.int32),
            pltpu.VMEM((CE,), jnp.int32),
            pltpu.VMEM((CB,), jnp.int32),
            pltpu.VMEM((CB,), jnp.int32),
            pltpu.VMEM((GB, D), jnp.float32),
            pltpu.SemaphoreType.DMA,
        ],
    )


BM = 1000


def _tc_mlp_body(x_ref, seg_ref, wt_ref, b_ref, o_ref):
    x = x_ref[...]
    seg = seg_ref[...]
    xj = jnp.where(seg == -jnp.inf, jnp.float32(0), seg - x)
    wt = wt_ref[...]
    acc = jnp.dot(x, wt[:D], preferred_element_type=jnp.float32)
    acc = acc + jnp.dot(xj, wt[D:], preferred_element_type=jnp.float32)
    o_ref[...] = jnp.maximum(acc + b_ref[...], jnp.float32(0))


_tc_mlp = pl.pallas_call(
    _tc_mlp_body,
    grid=(N // BM,),
    in_specs=[
        pl.BlockSpec((BM, D), lambda i: (i, 0)),
        pl.BlockSpec((BM, D), lambda i: (i, 0)),
        pl.BlockSpec((2 * D, D), lambda i: (0, 0)),
        pl.BlockSpec((1, D), lambda i: (0, 0)),
    ],
    out_specs=pl.BlockSpec((BM, D), lambda i: (i, 0)),
    out_shape=jax.ShapeDtypeStruct((N, D), jnp.float32),
)


@jax.jit
def kernel(x, edge_index, W, b):
    src = edge_index[0]
    dst = edge_index[1]
    segmax = _sc_segmax()(x, src, dst)[:N]
    return _tc_mlp(x, segmax, W.T, b.reshape(1, D))


# speedup vs baseline: 1.2480x; 1.2480x over previous
import functools

import jax
import jax.numpy as jnp
from jax import lax
from jax.experimental import pallas as pl
from jax.experimental.pallas import tpu as pltpu
from jax.experimental.pallas import tpu_sc as plsc

N = 10000
E = 160000
D = 256
NC = 2
NS = 16
NW = NC * NS
L = 16

R = 320
NPAD = NW * R
CE = 2000
NCH = E // CE
GB = 48
CB = 2016
NF = D // L


def _sc_segmax_body(x_hbm, src_hbm, dst_hbm, out_hbm,
                    acc, dstcA, srccA, dstcB, srccB,
                    srcbA, dlbA, srcbB, dlbB, rows0, rows1,
                    semdA, semsA, semdB, semsB, semg0, semg1):
    wid = lax.axis_index("s") * NC + lax.axis_index("c")
    lo = wid * R

    neg = jnp.full((L,), -jnp.inf, dtype=jnp.float32)
    zi = jnp.zeros((L,), dtype=jnp.int32)

    @pl.loop(0, R)
    def _(r):
        for f in range(NF):
            acc[r, pl.ds(f * L, L)] = neg

    @pl.loop(0, CB // L)
    def _(i):
        srcbA[pl.ds(i * L, L)] = zi
        srcbB[pl.ds(i * L, L)] = zi

    def start_chunk(c, dstc, srcc, semd, sems):
        pltpu.async_copy(dst_hbm.at[pl.ds(c * CE, CE)], dstc, semd)
        pltpu.async_copy(src_hbm.at[pl.ds(c * CE, CE)], srcc, sems)

    def wait_chunk(c, dstc, srcc, semd, sems):
        pltpu.make_async_copy(dst_hbm.at[pl.ds(c * CE, CE)], dstc, semd).wait()
        pltpu.make_async_copy(src_hbm.at[pl.ds(c * CE, CE)], srcc, sems).wait()

    def filter_chunk(dstc, srcc, srcb, dlb):
        def filt(it, pos):
            dstv = dstc[pl.ds(it * L, L)]
            srcv = srcc[pl.ds(it * L, L)]
            mask = (dstv >= lo) & (dstv < lo + R)
            mi = jnp.where(mask, 1, 0).astype(jnp.int32)
            posv = pos + plsc.cumsum(mi) - mi
            plsc.store_scatter(srcb, [posv], srcv, mask=mask)
            plsc.store_scatter(dlb, [posv], dstv - lo, mask=mask)
            return pos + plsc.all_reduce_population_count(mask)

        pos = lax.fori_loop(0, CE // L, filt, jnp.zeros((L,), jnp.int32))
        return jnp.max(pos)

    def start_g(srcb, b, rows, sem):
        pltpu.async_copy(x_hbm.at[srcb.at[pl.ds(b * GB, GB)]], rows, sem)

    def wait_g(srcb, b, rows, sem):
        pltpu.make_async_copy(
            x_hbm.at[srcb.at[pl.ds(b * GB, GB)]], rows, sem).wait()

    def upd_batch(dlb, rows, b, cnt):
        def upd(i, _):
            dl = dlb[pl.ds(b * GB + i, L)][0]
            for f in range(NF):
                s = pl.ds(f * L, L)
                acc[dl, s] = jnp.maximum(acc[dl, s], rows[i, s])
            return 0

        lax.fori_loop(0, cnt, upd, 0)

    def update_chunk(srcb, dlb, m):
        nb = jnp.maximum((m + GB - 1) // GB, 1)

        def pair(k, _):
            b0 = 2 * k

            @pl.when(b0 + 1 < nb)
            def _():
                start_g(srcb, b0 + 1, rows1, semg1)

            wait_g(srcb, b0, rows0, semg0)
            upd_batch(dlb, rows0, b0, jnp.minimum(GB, m - b0 * GB))

            @pl.when(b0 + 1 < nb)
            def _():
                @pl.when(b0 + 2 < nb)
                def _():
                    start_g(srcb, b0 + 2, rows0, semg0)

                wait_g(srcb, b0 + 1, rows1, semg1)
                upd_batch(dlb, rows1, b0 + 1, jnp.minimum(GB, m - (b0 + 1) * GB))

            return 0

        lax.fori_loop(0, (nb + 1) // 2, pair, 0)

    bufs = (
        (srcbA, dlbA, dstcA, srccA, semdA, semsA),
        (srcbB, dlbB, dstcB, srccB, semdB, semsB),
    )

    def stage(c, m_cur, par, do_prefetch, do_filter):
        srcb0, dlb0 = bufs[par][0], bufs[par][1]
        srcb1, dlb1 = bufs[1 - par][0], bufs[1 - par][1]
        dstc1, srcc1, semd1, sems1 = bufs[1 - par][2:6]
        dstc0, srcc0, semd0, sems0 = bufs[par][2:6]

        start_g(srcb0, 0, rows0, semg0)
        if do_filter:
            wait_chunk(c + 1, dstc1, srcc1, semd1, sems1)
            if do_prefetch:
                start_chunk(c + 2, dstc0, srcc0, semd0, sems0)
            m_next = filter_chunk(dstc1, srcc1, srcb1, dlb1)
        else:
            m_next = jnp.int32(0)
        update_chunk(srcb0, dlb0, m_cur)
        return m_next

    pltpu.sync_copy(dst_hbm.at[pl.ds(0, CE)], dstcA)
    pltpu.sync_copy(src_hbm.at[pl.ds(0, CE)], srccA)
    start_chunk(1, dstcB, srccB, semdB, semsB)
    m0 = filter_chunk(dstcA, srccA, srcbA, dlbA)

    def body2(i, m_cur):
        c = 2 * i
        m1 = stage(c, m_cur, 0, True, True)
        return stage(c + 1, m1, 1, True, True)

    m = lax.fori_loop(0, (NCH - 2) // 2, body2, m0)
    m = stage(NCH - 2, m, 0, False, True)
    stage(NCH - 1, m, 1, False, False)

    pltpu.sync_copy(acc, out_hbm.at[pl.ds(lo, R)])


@functools.cache
def _sc_segmax():
    return pl.kernel(
        _sc_segmax_body,
        out_type=jax.ShapeDtypeStruct((NPAD, D), jnp.float32),
        mesh=plsc.VectorSubcoreMesh(
            core_axis_name="c", subcore_axis_name="s",
            num_cores=NC, num_subcores=NS),
        compiler_params=pltpu.CompilerParams(needs_layout_passes=False),
        scratch_types=[
            pltpu.VMEM((R, D), jnp.float32),
            pltpu.VMEM((CE,), jnp.int32),
            pltpu.VMEM((CE,), jnp.int32),
            pltpu.VMEM((CE,), jnp.int32),
            pltpu.VMEM((CE,), jnp.int32),
            pltpu.VMEM((CB,), jnp.int32),
            pltpu.VMEM((CB,), jnp.int32),
            pltpu.VMEM((CB,), jnp.int32),
            pltpu.VMEM((CB,), jnp.int32),
            pltpu.VMEM((GB, D), jnp.float32),
            pltpu.VMEM((GB, D), jnp.float32),
            pltpu.SemaphoreType.DMA,
            pltpu.SemaphoreType.DMA,
            pltpu.SemaphoreType.DMA,
            pltpu.SemaphoreType.DMA,
            pltpu.SemaphoreType.DMA,
            pltpu.SemaphoreType.DMA,
        ],
    )


BM = 1000


def _tc_mlp_body(x_ref, seg_ref, wt_ref, b_ref, o_ref):
    x = x_ref[...]
    seg = seg_ref[...]
    xj = jnp.where(seg == -jnp.inf, jnp.float32(0), seg - x)
    wt = wt_ref[...]
    acc = jnp.dot(x, wt[:D], preferred_element_type=jnp.float32)
    acc = acc + jnp.dot(xj, wt[D:], preferred_element_type=jnp.float32)
    o_ref[...] = jnp.maximum(acc + b_ref[...], jnp.float32(0))


_tc_mlp = pl.pallas_call(
    _tc_mlp_body,
    grid=(N // BM,),
    in_specs=[
        pl.BlockSpec((BM, D), lambda i: (i, 0)),
        pl.BlockSpec((BM, D), lambda i: (i, 0)),
        pl.BlockSpec((2 * D, D), lambda i: (0, 0)),
        pl.BlockSpec((1, D), lambda i: (0, 0)),
    ],
    out_specs=pl.BlockSpec((BM, D), lambda i: (i, 0)),
    out_shape=jax.ShapeDtypeStruct((N, D), jnp.float32),
)


@jax.jit
def kernel(x, edge_index, W, b):
    src = edge_index[0]
    dst = edge_index[1]
    segmax = _sc_segmax()(x, src, dst)[:N]
    return _tc_mlp(x, segmax, W.T, b.reshape(1, D))


# speedup vs baseline: 1.8157x; 1.4548x over previous
import functools

import jax
import jax.numpy as jnp
from jax import lax
from jax.experimental import pallas as pl
from jax.experimental.pallas import tpu as pltpu
from jax.experimental.pallas import tpu_sc as plsc

N = 10000
E = 160000
D = 256
NC = 2
NS = 16
NW = NC * NS
L = 16
LB = 32

R = 320
NPAD = NW * R
CE = 4000
NCH = E // CE
GB = 64
CB = 4032
NFB = D // LB


def _sc_segmax_body(x_hbm, src_hbm, dst_hbm, out_hbm,
                    acc, dstcA, srccA, dstcB, srccB,
                    srcbA, dlbA, srcbB, dlbB, rows0, rows1,
                    semdA, semsA, semdB, semsB, semg0, semg1):
    wid = lax.axis_index("s") * NC + lax.axis_index("c")
    lo = wid * R

    negi = jnp.full((L,), jnp.uint32(0xFF80FF80).astype(jnp.int32),
                    dtype=jnp.int32)
    zi = jnp.zeros((L,), dtype=jnp.int32)

    @pl.loop(0, R)
    def _(r):
        for f in range(NFB):
            acc[r, pl.ds(f * L, L)] = negi

    @pl.loop(0, CB // L)
    def _(i):
        srcbA[pl.ds(i * L, L)] = zi
        srcbB[pl.ds(i * L, L)] = zi

    def start_chunk(c, dstc, srcc, semd, sems):
        pltpu.async_copy(dst_hbm.at[pl.ds(c * CE, CE)], dstc, semd)
        pltpu.async_copy(src_hbm.at[pl.ds(c * CE, CE)], srcc, sems)

    def wait_chunk(c, dstc, srcc, semd, sems):
        pltpu.make_async_copy(dst_hbm.at[pl.ds(c * CE, CE)], dstc, semd).wait()
        pltpu.make_async_copy(src_hbm.at[pl.ds(c * CE, CE)], srcc, sems).wait()

    def filter_chunk(dstc, srcc, srcb, dlb):
        def filt(it, pos):
            dstv = dstc[pl.ds(it * L, L)]
            srcv = srcc[pl.ds(it * L, L)]
            mask = (dstv >= lo) & (dstv < lo + R)
            mi = jnp.where(mask, 1, 0).astype(jnp.int32)
            posv = pos + plsc.cumsum(mi) - mi
            plsc.store_scatter(srcb, [posv], srcv, mask=mask)
            plsc.store_scatter(dlb, [posv], dstv - lo, mask=mask)
            return pos + plsc.all_reduce_population_count(mask)

        pos = lax.fori_loop(0, CE // L, filt, jnp.zeros((L,), jnp.int32))
        return jnp.max(pos)

    def start_g(srcb, b, rows, sem):
        pltpu.async_copy(x_hbm.at[srcb.at[pl.ds(b * GB, GB)]], rows, sem)

    def wait_g(srcb, b, rows, sem):
        pltpu.make_async_copy(
            x_hbm.at[srcb.at[pl.ds(b * GB, GB)]], rows, sem).wait()

    def upd_batch(dlb, rows, b, cnt):
        def upd(i, _):
            dl = dlb[pl.ds(b * GB + i, L)][0]
            for f in range(NFB):
                s = pl.ds(f * L, L)
                a = plsc.bitcast(acc[dl, s], jnp.bfloat16)
                r = plsc.bitcast(rows[i, s], jnp.bfloat16)
                acc[dl, s] = plsc.bitcast(jnp.maximum(a, r), jnp.int32)
            return 0

        lax.fori_loop(0, cnt, upd, 0)

    def update_chunk(srcb, dlb, m):
        nb = jnp.maximum((m + GB - 1) // GB, 1)

        def pair(k, _):
            b0 = 2 * k

            @pl.when(b0 + 1 < nb)
            def _():
                start_g(srcb, b0 + 1, rows1, semg1)

            wait_g(srcb, b0, rows0, semg0)
            upd_batch(dlb, rows0, b0, jnp.minimum(GB, m - b0 * GB))

            @pl.when(b0 + 1 < nb)
            def _():
                @pl.when(b0 + 2 < nb)
                def _():
                    start_g(srcb, b0 + 2, rows0, semg0)

                wait_g(srcb, b0 + 1, rows1, semg1)
                upd_batch(dlb, rows1, b0 + 1, jnp.minimum(GB, m - (b0 + 1) * GB))

            return 0

        lax.fori_loop(0, (nb + 1) // 2, pair, 0)

    bufs = (
        (srcbA, dlbA, dstcA, srccA, semdA, semsA),
        (srcbB, dlbB, dstcB, srccB, semdB, semsB),
    )

    def stage(c, m_cur, par, do_prefetch, do_filter):
        srcb0, dlb0 = bufs[par][0], bufs[par][1]
        srcb1, dlb1 = bufs[1 - par][0], bufs[1 - par][1]
        dstc1, srcc1, semd1, sems1 = bufs[1 - par][2:6]
        dstc0, srcc0, semd0, sems0 = bufs[par][2:6]

        start_g(srcb0, 0, rows0, semg0)
        if do_filter:
            wait_chunk(c + 1, dstc1, srcc1, semd1, sems1)
            if do_prefetch:
                start_chunk(c + 2, dstc0, srcc0, semd0, sems0)
            m_next = filter_chunk(dstc1, srcc1, srcb1, dlb1)
        else:
            m_next = jnp.int32(0)
        update_chunk(srcb0, dlb0, m_cur)
        return m_next

    pltpu.sync_copy(dst_hbm.at[pl.ds(0, CE)], dstcA)
    pltpu.sync_copy(src_hbm.at[pl.ds(0, CE)], srccA)
    start_chunk(1, dstcB, srccB, semdB, semsB)
    m0 = filter_chunk(dstcA, srccA, srcbA, dlbA)

    def body2(i, m_cur):
        c = 2 * i
        m1 = stage(c, m_cur, 0, True, True)
        return stage(c + 1, m1, 1, True, True)

    m = lax.fori_loop(0, (NCH - 2) // 2, body2, m0)
    m = stage(NCH - 2, m, 0, False, True)
    stage(NCH - 1, m, 1, False, False)

    pltpu.sync_copy(acc, out_hbm.at[pl.ds(lo, R)])


@functools.cache
def _sc_segmax():
    return pl.kernel(
        _sc_segmax_body,
        out_type=jax.ShapeDtypeStruct((NPAD, D // 2), jnp.int32),
        mesh=plsc.VectorSubcoreMesh(
            core_axis_name="c", subcore_axis_name="s",
            num_cores=NC, num_subcores=NS),
        compiler_params=pltpu.CompilerParams(needs_layout_passes=False),
        scratch_types=[
            pltpu.VMEM((R, D // 2), jnp.int32),
            pltpu.VMEM((CE,), jnp.int32),
            pltpu.VMEM((CE,), jnp.int32),
            pltpu.VMEM((CE,), jnp.int32),
            pltpu.VMEM((CE,), jnp.int32),
            pltpu.VMEM((CB,), jnp.int32),
            pltpu.VMEM((CB,), jnp.int32),
            pltpu.VMEM((CB,), jnp.int32),
            pltpu.VMEM((CB,), jnp.int32),
            pltpu.VMEM((GB, D // 2), jnp.int32),
            pltpu.VMEM((GB, D // 2), jnp.int32),
            pltpu.SemaphoreType.DMA,
            pltpu.SemaphoreType.DMA,
            pltpu.SemaphoreType.DMA,
            pltpu.SemaphoreType.DMA,
            pltpu.SemaphoreType.DMA,
            pltpu.SemaphoreType.DMA,
        ],
    )


BM = 1000


def _tc_mlp_body(x_ref, seg_ref, wt_ref, b_ref, o_ref):
    x = x_ref[...]
    seg = seg_ref[...].astype(jnp.float32)
    xj = jnp.where(seg == -jnp.inf, jnp.float32(0), seg - x)
    wt = wt_ref[...]
    acc = jnp.dot(x, wt[:D], preferred_element_type=jnp.float32)
    acc = acc + jnp.dot(xj, wt[D:], preferred_element_type=jnp.float32)
    o_ref[...] = jnp.maximum(acc + b_ref[...], jnp.float32(0))


_tc_mlp = pl.pallas_call(
    _tc_mlp_body,
    grid=(N // BM,),
    in_specs=[
        pl.BlockSpec((BM, D), lambda i: (i, 0)),
        pl.BlockSpec((BM, D), lambda i: (i, 0)),
        pl.BlockSpec((2 * D, D), lambda i: (0, 0)),
        pl.BlockSpec((1, D), lambda i: (0, 0)),
    ],
    out_specs=pl.BlockSpec((BM, D), lambda i: (i, 0)),
    out_shape=jax.ShapeDtypeStruct((N, D), jnp.float32),
)


@jax.jit
def kernel(x, edge_index, W, b):
    src = edge_index[0]
    dst = edge_index[1]
    xb = x.astype(jnp.bfloat16)
    xi = lax.bitcast_convert_type(xb.reshape(N, D // 2, 2), jnp.int32)
    seg_i = _sc_segmax()(xi, src, dst)
    segmax = lax.bitcast_convert_type(seg_i, jnp.bfloat16).reshape(NPAD, D)[:N]
    return _tc_mlp(x, segmax, W.T, b.reshape(1, D))


# speedup vs baseline: 2.8442x; 1.5665x over previous
import functools

import jax
import jax.numpy as jnp
from jax import lax
from jax.experimental import pallas as pl
from jax.experimental.pallas import tpu as pltpu
from jax.experimental.pallas import tpu_sc as plsc

N = 10000
E = 160000
D = 256
NC = 2
NS = 16
NW = NC * NS
L = 16
LB = 32

R = 640
NPAD = NS * R
E2 = E // 2
CE = 4000
NCH = E2 // CE
GB = 64
CB = 4032
NFB = D // LB


def _sc_segmax_body(x_hbm, src_hbm, dst_hbm, out_hbm,
                    acc, dstcA, srccA, dstcB, srccB,
                    srcbA, dlbA, srcbB, dlbB, rows0, rows1,
                    semdA, semsA, semdB, semsB, semg0, semg1):
    sid = lax.axis_index("s")
    cid = lax.axis_index("c")
    lo = sid * R
    ebase = cid * E2

    negi = jnp.full((L,), jnp.uint32(0xFF80FF80).astype(jnp.int32),
                    dtype=jnp.int32)
    zi = jnp.zeros((L,), dtype=jnp.int32)

    @pl.loop(0, R)
    def _(r):
        for f in range(NFB):
            acc[r, pl.ds(f * L, L)] = negi

    @pl.loop(0, CB // L)
    def _(i):
        srcbA[pl.ds(i * L, L)] = zi
        srcbB[pl.ds(i * L, L)] = zi

    def start_chunk(c, dstc, srcc, semd, sems):
        pltpu.async_copy(dst_hbm.at[pl.ds(ebase + c * CE, CE)], dstc, semd)
        pltpu.async_copy(src_hbm.at[pl.ds(ebase + c * CE, CE)], srcc, sems)

    def wait_chunk(c, dstc, srcc, semd, sems):
        pltpu.make_async_copy(
            dst_hbm.at[pl.ds(ebase + c * CE, CE)], dstc, semd).wait()
        pltpu.make_async_copy(
            src_hbm.at[pl.ds(ebase + c * CE, CE)], srcc, sems).wait()

    def filter_chunk(dstc, srcc, srcb, dlb):
        def filt(it, pos):
            dstv = dstc[pl.ds(it * L, L)]
            srcv = srcc[pl.ds(it * L, L)]
            mask = (dstv >= lo) & (dstv < lo + R)
            mi = jnp.where(mask, 1, 0).astype(jnp.int32)
            posv = pos + plsc.cumsum(mi) - mi
            plsc.store_scatter(srcb, [posv], srcv, mask=mask)
            plsc.store_scatter(dlb, [posv], dstv - lo, mask=mask)
            return pos + plsc.all_reduce_population_count(mask)

        pos = lax.fori_loop(0, CE // L, filt, jnp.zeros((L,), jnp.int32))
        return jnp.max(pos)

    def start_g(srcb, b, rows, sem):
        pltpu.async_copy(x_hbm.at[srcb.at[pl.ds(b * GB, GB)]], rows, sem)

    def wait_g(srcb, b, rows, sem):
        pltpu.make_async_copy(
            x_hbm.at[srcb.at[pl.ds(b * GB, GB)]], rows, sem).wait()

    def upd_batch(dlb, rows, b, cnt):
        def upd(i, _):
            dl = dlb[pl.ds(b * GB + i, L)][0]
            for f in range(NFB):
                s = pl.ds(f * L, L)
                a = plsc.bitcast(acc[dl, s], jnp.bfloat16)
                r = plsc.bitcast(rows[i, s], jnp.bfloat16)
                acc[dl, s] = plsc.bitcast(jnp.maximum(a, r), jnp.int32)
            return 0

        lax.fori_loop(0, cnt, upd, 0)

    def update_chunk(srcb, dlb, m):
        nb = jnp.maximum((m + GB - 1) // GB, 1)

        def pair(k, _):
            b0 = 2 * k

            @pl.when(b0 + 1 < nb)
            def _():
                start_g(srcb, b0 + 1, rows1, semg1)

            wait_g(srcb, b0, rows0, semg0)
            upd_batch(dlb, rows0, b0, jnp.minimum(GB, m - b0 * GB))

            @pl.when(b0 + 1 < nb)
            def _():
                @pl.when(b0 + 2 < nb)
                def _():
                    start_g(srcb, b0 + 2, rows0, semg0)

                wait_g(srcb, b0 + 1, rows1, semg1)
                upd_batch(dlb, rows1, b0 + 1, jnp.minimum(GB, m - (b0 + 1) * GB))

            return 0

        lax.fori_loop(0, (nb + 1) // 2, pair, 0)

    bufs = (
        (srcbA, dlbA, dstcA, srccA, semdA, semsA),
        (srcbB, dlbB, dstcB, srccB, semdB, semsB),
    )

    def stage(c, m_cur, par, do_prefetch, do_filter):
        srcb0, dlb0 = bufs[par][0], bufs[par][1]
        srcb1, dlb1 = bufs[1 - par][0], bufs[1 - par][1]
        dstc1, srcc1, semd1, sems1 = bufs[1 - par][2:6]
        dstc0, srcc0, semd0, sems0 = bufs[par][2:6]

        start_g(srcb0, 0, rows0, semg0)
        if do_filter:
            wait_chunk(c + 1, dstc1, srcc1, semd1, sems1)
            if do_prefetch:
                start_chunk(c + 2, dstc0, srcc0, semd0, sems0)
            m_next = filter_chunk(dstc1, srcc1, srcb1, dlb1)
        else:
            m_next = jnp.int32(0)
        update_chunk(srcb0, dlb0, m_cur)
        return m_next

    pltpu.sync_copy(dst_hbm.at[pl.ds(ebase, CE)], dstcA)
    pltpu.sync_copy(src_hbm.at[pl.ds(ebase, CE)], srccA)
    start_chunk(1, dstcB, srccB, semdB, semsB)
    m0 = filter_chunk(dstcA, srccA, srcbA, dlbA)

    def body2(i, m_cur):
        c = 2 * i
        m1 = stage(c, m_cur, 0, True, True)
        return stage(c + 1, m1, 1, True, True)

    m = lax.fori_loop(0, (NCH - 2) // 2, body2, m0)
    m = stage(NCH - 2, m, 0, False, True)
    stage(NCH - 1, m, 1, False, False)

    pltpu.sync_copy(acc, out_hbm.at[pl.ds(cid * NPAD + lo, R)])


@functools.cache
def _sc_segmax():
    return pl.kernel(
        _sc_segmax_body,
        out_type=jax.ShapeDtypeStruct((2 * NPAD, D // 2), jnp.int32),
        mesh=plsc.VectorSubcoreMesh(
            core_axis_name="c", subcore_axis_name="s",
            num_cores=NC, num_subcores=NS),
        compiler_params=pltpu.CompilerParams(needs_layout_passes=False),
        scratch_types=[
            pltpu.VMEM((R, D // 2), jnp.int32),
            pltpu.VMEM((CE,), jnp.int32),
            pltpu.VMEM((CE,), jnp.int32),
            pltpu.VMEM((CE,), jnp.int32),
            pltpu.VMEM((CE,), jnp.int32),
            pltpu.VMEM((CB,), jnp.int32),
            pltpu.VMEM((CB,), jnp.int32),
            pltpu.VMEM((CB,), jnp.int32),
            pltpu.VMEM((CB,), jnp.int32),
            pltpu.VMEM((GB, D // 2), jnp.int32),
            pltpu.VMEM((GB, D // 2), jnp.int32),
            pltpu.SemaphoreType.DMA,
            pltpu.SemaphoreType.DMA,
            pltpu.SemaphoreType.DMA,
            pltpu.SemaphoreType.DMA,
            pltpu.SemaphoreType.DMA,
            pltpu.SemaphoreType.DMA,
        ],
    )


BM = 1000


def _tc_mlp_body(x_ref, seg0_ref, seg1_ref, wt_ref, b_ref, o_ref):
    x = x_ref[...]
    seg = jnp.maximum(seg0_ref[...], seg1_ref[...]).astype(jnp.float32)
    xj = jnp.where(seg == -jnp.inf, jnp.float32(0), seg - x)
    wt = wt_ref[...]
    acc = jnp.dot(x, wt[:D], preferred_element_type=jnp.float32)
    acc = acc + jnp.dot(xj, wt[D:], preferred_element_type=jnp.float32)
    o_ref[...] = jnp.maximum(acc + b_ref[...], jnp.float32(0))


_tc_mlp = pl.pallas_call(
    _tc_mlp_body,
    grid=(N // BM,),
    in_specs=[
        pl.BlockSpec((BM, D), lambda i: (i, 0)),
        pl.BlockSpec((BM, D), lambda i: (i, 0)),
        pl.BlockSpec((BM, D), lambda i: (i, 0)),
        pl.BlockSpec((2 * D, D), lambda i: (0, 0)),
        pl.BlockSpec((1, D), lambda i: (0, 0)),
    ],
    out_specs=pl.BlockSpec((BM, D), lambda i: (i, 0)),
    out_shape=jax.ShapeDtypeStruct((N, D), jnp.float32),
)


@jax.jit
def kernel(x, edge_index, W, b):
    src = edge_index[0]
    dst = edge_index[1]
    xb = x.astype(jnp.bfloat16)
    xi = lax.bitcast_convert_type(xb.reshape(N, D // 2, 2), jnp.int32)
    seg_i = _sc_segmax()(xi, src, dst)
    seg = lax.bitcast_convert_type(seg_i, jnp.bfloat16).reshape(2 * NPAD, D)
    return _tc_mlp(x, seg[:N], seg[NPAD:NPAD + N], W.T, b.reshape(1, D))


# speedup vs baseline: 2.8489x; 1.0017x over previous
import functools

import jax
import jax.numpy as jnp
from jax import lax
from jax.experimental import pallas as pl
from jax.experimental.pallas import tpu as pltpu
from jax.experimental.pallas import tpu_sc as plsc

N = 10000
E = 160000
D = 256
NC = 2
NS = 16
NW = NC * NS
L = 16
LB = 32

R = 640
NPAD = NS * R
E2 = E // 2
CE = 4000
NCH = E2 // CE
GB = 64
CB = 4032
NFB = D // LB


def _sc_segmax_body(x_hbm, src_hbm, dst_hbm, out_hbm,
                    acc, dstcA, srccA, dstcB, srccB,
                    srcbA, dlbA, srcbB, dlbB, rows0, rows1,
                    semdA, semsA, semdB, semsB, semg0, semg1):
    sid = lax.axis_index("s")
    cid = lax.axis_index("c")
    lo = sid * R
    ebase = cid * E2

    negi = jnp.full((L,), jnp.uint32(0xFF80FF80).astype(jnp.int32),
                    dtype=jnp.int32)
    zi = jnp.zeros((L,), dtype=jnp.int32)

    @pl.loop(0, R)
    def _(r):
        for f in range(NFB):
            acc[r, pl.ds(f * L, L)] = negi

    @pl.loop(0, CB // L)
    def _(i):
        srcbA[pl.ds(i * L, L)] = zi
        srcbB[pl.ds(i * L, L)] = zi

    def start_chunk(c, dstc, srcc, semd, sems):
        pltpu.async_copy(dst_hbm.at[pl.ds(ebase + c * CE, CE)], dstc, semd)
        pltpu.async_copy(src_hbm.at[pl.ds(ebase + c * CE, CE)], srcc, sems)

    def wait_chunk(c, dstc, srcc, semd, sems):
        pltpu.make_async_copy(
            dst_hbm.at[pl.ds(ebase + c * CE, CE)], dstc, semd).wait()
        pltpu.make_async_copy(
            src_hbm.at[pl.ds(ebase + c * CE, CE)], srcc, sems).wait()

    def filter_chunk(dstc, srcc, srcb, dlb):
        def filt(it, pos):
            for u in range(2):
                o = (2 * it + u) * L
                dstv = dstc[pl.ds(o, L)]
                srcv = srcc[pl.ds(o, L)]
                mask = (dstv >= lo) & (dstv < lo + R)
                mi = jnp.where(mask, 1, 0).astype(jnp.int32)
                posv = pos + plsc.cumsum(mi) - mi
                plsc.store_scatter(srcb, [posv], srcv, mask=mask)
                plsc.store_scatter(dlb, [posv], dstv - lo, mask=mask)
                pos = pos + plsc.all_reduce_population_count(mask)
            return pos

        pos = lax.fori_loop(0, CE // L // 2, filt, jnp.zeros((L,), jnp.int32))
        return jnp.max(pos)

    def start_g(srcb, b, rows, sem):
        pltpu.async_copy(x_hbm.at[srcb.at[pl.ds(b * GB, GB)]], rows, sem)

    def wait_g(srcb, b, rows, sem):
        pltpu.make_async_copy(
            x_hbm.at[srcb.at[pl.ds(b * GB, GB)]], rows, sem).wait()

    def _upd_one(dl, rows, i):
        for f in range(NFB):
            s = pl.ds(f * L, L)
            a = plsc.bitcast(acc[dl, s], jnp.bfloat16)
            r = plsc.bitcast(rows[i, s], jnp.bfloat16)
            acc[dl, s] = plsc.bitcast(jnp.maximum(a, r), jnp.int32)

    def upd_batch(dlb, rows, b, cnt):
        ng = cnt // L

        def grp(k, _):
            dlv = dlb[pl.ds(b * GB + k * L, L)]
            for j in range(L):
                _upd_one(dlv[j], rows, k * L + j)
            return 0

        lax.fori_loop(0, ng, grp, 0)

        def tail(i, _):
            _upd_one(dlb[pl.ds(b * GB + ng * L + i, L)][0], rows, ng * L + i)
            return 0

        lax.fori_loop(0, cnt - ng * L, tail, 0)

    def update_chunk(srcb, dlb, m):
        nb = jnp.maximum((m + GB - 1) // GB, 1)

        def pair(k, _):
            b0 = 2 * k

            @pl.when(b0 + 1 < nb)
            def _():
                start_g(srcb, b0 + 1, rows1, semg1)

            wait_g(srcb, b0, rows0, semg0)
            upd_batch(dlb, rows0, b0, jnp.minimum(GB, m - b0 * GB))

            @pl.when(b0 + 1 < nb)
            def _():
                @pl.when(b0 + 2 < nb)
                def _():
                    start_g(srcb, b0 + 2, rows0, semg0)

                wait_g(srcb, b0 + 1, rows1, semg1)
                upd_batch(dlb, rows1, b0 + 1, jnp.minimum(GB, m - (b0 + 1) * GB))

            return 0

        lax.fori_loop(0, (nb + 1) // 2, pair, 0)

    bufs = (
        (srcbA, dlbA, dstcA, srccA, semdA, semsA),
        (srcbB, dlbB, dstcB, srccB, semdB, semsB),
    )

    def stage(c, m_cur, par, do_prefetch, do_filter):
        srcb0, dlb0 = bufs[par][0], bufs[par][1]
        srcb1, dlb1 = bufs[1 - par][0], bufs[1 - par][1]
        dstc1, srcc1, semd1, sems1 = bufs[1 - par][2:6]
        dstc0, srcc0, semd0, sems0 = bufs[par][2:6]

        start_g(srcb0, 0, rows0, semg0)
        if do_filter:
            wait_chunk(c + 1, dstc1, srcc1, semd1, sems1)
            if do_prefetch:
                start_chunk(c + 2, dstc0, srcc0, semd0, sems0)
            m_next = filter_chunk(dstc1, srcc1, srcb1, dlb1)
        else:
            m_next = jnp.int32(0)
        update_chunk(srcb0, dlb0, m_cur)
        return m_next

    pltpu.sync_copy(dst_hbm.at[pl.ds(ebase, CE)], dstcA)
    pltpu.sync_copy(src_hbm.at[pl.ds(ebase, CE)], srccA)
    start_chunk(1, dstcB, srccB, semdB, semsB)
    m0 = filter_chunk(dstcA, srccA, srcbA, dlbA)

    def body2(i, m_cur):
        c = 2 * i
        m1 = stage(c, m_cur, 0, True, True)
        return stage(c + 1, m1, 1, True, True)

    m = lax.fori_loop(0, (NCH - 2) // 2, body2, m0)
    m = stage(NCH - 2, m, 0, False, True)
    stage(NCH - 1, m, 1, False, False)

    pltpu.sync_copy(acc, out_hbm.at[pl.ds(cid * NPAD + lo, R)])


@functools.cache
def _sc_segmax():
    return pl.kernel(
        _sc_segmax_body,
        out_type=jax.ShapeDtypeStruct((2 * NPAD, D // 2), jnp.int32),
        mesh=plsc.VectorSubcoreMesh(
            core_axis_name="c", subcore_axis_name="s",
            num_cores=NC, num_subcores=NS),
        compiler_params=pltpu.CompilerParams(needs_layout_passes=False),
        scratch_types=[
            pltpu.VMEM((R, D // 2), jnp.int32),
            pltpu.VMEM((CE,), jnp.int32),
            pltpu.VMEM((CE,), jnp.int32),
            pltpu.VMEM((CE,), jnp.int32),
            pltpu.VMEM((CE,), jnp.int32),
            pltpu.VMEM((CB,), jnp.int32),
            pltpu.VMEM((CB,), jnp.int32),
            pltpu.VMEM((CB,), jnp.int32),
            pltpu.VMEM((CB,), jnp.int32),
            pltpu.VMEM((GB, D // 2), jnp.int32),
            pltpu.VMEM((GB, D // 2), jnp.int32),
            pltpu.SemaphoreType.DMA,
            pltpu.SemaphoreType.DMA,
            pltpu.SemaphoreType.DMA,
            pltpu.SemaphoreType.DMA,
            pltpu.SemaphoreType.DMA,
            pltpu.SemaphoreType.DMA,
        ],
    )


BM = 1000


def _tc_mlp_body(x_ref, seg0_ref, seg1_ref, wt_ref, b_ref, o_ref):
    x = x_ref[...]
    seg = jnp.maximum(seg0_ref[...], seg1_ref[...]).astype(jnp.float32)
    xj = jnp.where(seg == -jnp.inf, jnp.float32(0), seg - x)
    wt = wt_ref[...]
    acc = jnp.dot(x, wt[:D], preferred_element_type=jnp.float32)
    acc = acc + jnp.dot(xj, wt[D:], preferred_element_type=jnp.float32)
    o_ref[...] = jnp.maximum(acc + b_ref[...], jnp.float32(0))


_tc_mlp = pl.pallas_call(
    _tc_mlp_body,
    grid=(N // BM,),
    in_specs=[
        pl.BlockSpec((BM, D), lambda i: (i, 0)),
        pl.BlockSpec((BM, D), lambda i: (i, 0)),
        pl.BlockSpec((BM, D), lambda i: (i, 0)),
        pl.BlockSpec((2 * D, D), lambda i: (0, 0)),
        pl.BlockSpec((1, D), lambda i: (0, 0)),
    ],
    out_specs=pl.BlockSpec((BM, D), lambda i: (i, 0)),
    out_shape=jax.ShapeDtypeStruct((N, D), jnp.float32),
)


@jax.jit
def kernel(x, edge_index, W, b):
    src = edge_index[0]
    dst = edge_index[1]
    xb = x.astype(jnp.bfloat16)
    xi = lax.bitcast_convert_type(xb.reshape(N, D // 2, 2), jnp.int32)
    seg_i = _sc_segmax()(xi, src, dst)
    seg = lax.bitcast_convert_type(seg_i, jnp.bfloat16).reshape(2 * NPAD, D)
    return _tc_mlp(x, seg[:N], seg[NPAD:NPAD + N], W.T, b.reshape(1, D))


# speedup vs baseline: 2.8556x; 1.0023x over previous
import functools

import jax
import jax.numpy as jnp
from jax import lax
from jax.experimental import pallas as pl
from jax.experimental.pallas import tpu as pltpu
from jax.experimental.pallas import tpu_sc as plsc

N = 10000
E = 160000
D = 256
NC = 2
NS = 16
NW = NC * NS
L = 16
LB = 32

R = 640
NPAD = NS * R
E2 = E // 2
CE = 4000
NCH = E2 // CE
GB = 64
CB = 4032
NFB = D // LB


def _sc_segmax_body(x_hbm, src_hbm, dst_hbm, out_hbm,
                    acc, dstcA, srccA, dstcB, srccB,
                    srcbA, dlbA, srcbB, dlbB, rows0, rows1,
                    semdA, semsA, semdB, semsB, semg0, semg1):
    sid = lax.axis_index("s")
    cid = lax.axis_index("c")
    lo = sid * R
    ebase = cid * E2

    negi = jnp.full((L,), jnp.uint32(0xFF80FF80).astype(jnp.int32),
                    dtype=jnp.int32)
    zi = jnp.zeros((L,), dtype=jnp.int32)

    @pl.loop(0, R)
    def _(r):
        for f in range(NFB):
            acc[r, pl.ds(f * L, L)] = negi

    @pl.loop(0, CB // L)
    def _(i):
        srcbA[pl.ds(i * L, L)] = zi
        srcbB[pl.ds(i * L, L)] = zi

    def start_chunk(c, dstc, srcc, semd, sems):
        pltpu.async_copy(dst_hbm.at[pl.ds(ebase + c * CE, CE)], dstc, semd)
        pltpu.async_copy(src_hbm.at[pl.ds(ebase + c * CE, CE)], srcc, sems)

    def wait_chunk(c, dstc, srcc, semd, sems):
        pltpu.make_async_copy(
            dst_hbm.at[pl.ds(ebase + c * CE, CE)], dstc, semd).wait()
        pltpu.make_async_copy(
            src_hbm.at[pl.ds(ebase + c * CE, CE)], srcc, sems).wait()

    def filter_chunk(dstc, srcc, srcb, dlb):
        def filt(it, pos):
            for u in range(2):
                o = (2 * it + u) * L
                dstv = dstc[pl.ds(o, L)]
                srcv = srcc[pl.ds(o, L)]
                mask = (dstv >= lo) & (dstv < lo + R)
                mi = jnp.where(mask, 1, 0).astype(jnp.int32)
                posv = pos + plsc.cumsum(mi) - mi
                plsc.store_scatter(srcb, [posv], srcv, mask=mask)
                plsc.store_scatter(dlb, [posv], dstv - lo, mask=mask)
                pos = pos + plsc.all_reduce_population_count(mask)
            return pos

        pos = lax.fori_loop(0, CE // L // 2, filt, jnp.zeros((L,), jnp.int32))
        return jnp.max(pos)

    def start_g(srcb, b, rows, sem):
        pltpu.async_copy(x_hbm.at[srcb.at[pl.ds(b * GB, GB)]], rows, sem)

    def wait_g(srcb, b, rows, sem):
        pltpu.make_async_copy(
            x_hbm.at[srcb.at[pl.ds(b * GB, GB)]], rows, sem).wait()

    def _upd_one(dl, rows, i):
        sls = [pl.ds(f * L, L) for f in range(NFB)]
        avs = [plsc.bitcast(acc[dl, s], jnp.bfloat16) for s in sls]
        rvs = [plsc.bitcast(rows[i, s], jnp.bfloat16) for s in sls]
        mvs = [plsc.bitcast(jnp.maximum(a, r), jnp.int32)
               for a, r in zip(avs, rvs)]
        for s, m in zip(sls, mvs):
            acc[dl, s] = m

    def upd_batch(dlb, rows, b, cnt):
        ng = cnt // L

        def grp(k, _):
            dlv = dlb[pl.ds(b * GB + k * L, L)]
            for j in range(L):
                _upd_one(dlv[j], rows, k * L + j)
            return 0

        lax.fori_loop(0, ng, grp, 0)

        def tail(i, _):
            _upd_one(dlb[pl.ds(b * GB + ng * L + i, L)][0], rows, ng * L + i)
            return 0

        lax.fori_loop(0, cnt - ng * L, tail, 0)

    def update_chunk(srcb, dlb, m):
        nb = jnp.maximum((m + GB - 1) // GB, 1)

        def pair(k, _):
            b0 = 2 * k

            @pl.when(b0 + 1 < nb)
            def _():
                start_g(srcb, b0 + 1, rows1, semg1)

            wait_g(srcb, b0, rows0, semg0)
            upd_batch(dlb, rows0, b0, jnp.minimum(GB, m - b0 * GB))

            @pl.when(b0 + 1 < nb)
            def _():
                @pl.when(b0 + 2 < nb)
                def _():
                    start_g(srcb, b0 + 2, rows0, semg0)

                wait_g(srcb, b0 + 1, rows1, semg1)
                upd_batch(dlb, rows1, b0 + 1, jnp.minimum(GB, m - (b0 + 1) * GB))

            return 0

        lax.fori_loop(0, (nb + 1) // 2, pair, 0)

    bufs = (
        (srcbA, dlbA, dstcA, srccA, semdA, semsA),
        (srcbB, dlbB, dstcB, srccB, semdB, semsB),
    )

    def stage(c, m_cur, par, do_prefetch, do_filter):
        srcb0, dlb0 = bufs[par][0], bufs[par][1]
        srcb1, dlb1 = bufs[1 - par][0], bufs[1 - par][1]
        dstc1, srcc1, semd1, sems1 = bufs[1 - par][2:6]
        dstc0, srcc0, semd0, sems0 = bufs[par][2:6]

        start_g(srcb0, 0, rows0, semg0)
        if do_filter:
            wait_chunk(c + 1, dstc1, srcc1, semd1, sems1)
            if do_prefetch:
                start_chunk(c + 2, dstc0, srcc0, semd0, sems0)
            m_next = filter_chunk(dstc1, srcc1, srcb1, dlb1)
        else:
            m_next = jnp.int32(0)
        update_chunk(srcb0, dlb0, m_cur)
        return m_next

    pltpu.sync_copy(dst_hbm.at[pl.ds(ebase, CE)], dstcA)
    pltpu.sync_copy(src_hbm.at[pl.ds(ebase, CE)], srccA)
    start_chunk(1, dstcB, srccB, semdB, semsB)
    m0 = filter_chunk(dstcA, srccA, srcbA, dlbA)

    def body2(i, m_cur):
        c = 2 * i
        m1 = stage(c, m_cur, 0, True, True)
        return stage(c + 1, m1, 1, True, True)

    m = lax.fori_loop(0, (NCH - 2) // 2, body2, m0)
    m = stage(NCH - 2, m, 0, False, True)
    stage(NCH - 1, m, 1, False, False)

    pltpu.sync_copy(acc, out_hbm.at[pl.ds(cid * NPAD + lo, R)])


@functools.cache
def _sc_segmax():
    return pl.kernel(
        _sc_segmax_body,
        out_type=jax.ShapeDtypeStruct((2 * NPAD, D // 2), jnp.int32),
        mesh=plsc.VectorSubcoreMesh(
            core_axis_name="c", subcore_axis_name="s",
            num_cores=NC, num_subcores=NS),
        compiler_params=pltpu.CompilerParams(needs_layout_passes=False),
        scratch_types=[
            pltpu.VMEM((R, D // 2), jnp.int32),
            pltpu.VMEM((CE,), jnp.int32),
            pltpu.VMEM((CE,), jnp.int32),
            pltpu.VMEM((CE,), jnp.int32),
            pltpu.VMEM((CE,), jnp.int32),
            pltpu.VMEM((CB,), jnp.int32),
            pltpu.VMEM((CB,), jnp.int32),
            pltpu.VMEM((CB,), jnp.int32),
            pltpu.VMEM((CB,), jnp.int32),
            pltpu.VMEM((GB, D // 2), jnp.int32),
            pltpu.VMEM((GB, D // 2), jnp.int32),
            pltpu.SemaphoreType.DMA,
            pltpu.SemaphoreType.DMA,
            pltpu.SemaphoreType.DMA,
            pltpu.SemaphoreType.DMA,
            pltpu.SemaphoreType.DMA,
            pltpu.SemaphoreType.DMA,
        ],
    )


BM = 1000


def _tc_mlp_body(x_ref, seg0_ref, seg1_ref, wt_ref, b_ref, o_ref):
    x = x_ref[...]
    seg = jnp.maximum(seg0_ref[...], seg1_ref[...]).astype(jnp.float32)
    xj = jnp.where(seg == -jnp.inf, jnp.float32(0), seg - x)
    wt = wt_ref[...]
    acc = jnp.dot(x, wt[:D], preferred_element_type=jnp.float32)
    acc = acc + jnp.dot(xj, wt[D:], preferred_element_type=jnp.float32)
    o_ref[...] = jnp.maximum(acc + b_ref[...], jnp.float32(0))


_tc_mlp = pl.pallas_call(
    _tc_mlp_body,
    grid=(N // BM,),
    in_specs=[
        pl.BlockSpec((BM, D), lambda i: (i, 0)),
        pl.BlockSpec((BM, D), lambda i: (i, 0)),
        pl.BlockSpec((BM, D), lambda i: (i, 0)),
        pl.BlockSpec((2 * D, D), lambda i: (0, 0)),
        pl.BlockSpec((1, D), lambda i: (0, 0)),
    ],
    out_specs=pl.BlockSpec((BM, D), lambda i: (i, 0)),
    out_shape=jax.ShapeDtypeStruct((N, D), jnp.float32),
)


@jax.jit
def kernel(x, edge_index, W, b):
    src = edge_index[0]
    dst = edge_index[1]
    xb = x.astype(jnp.bfloat16)
    xi = lax.bitcast_convert_type(xb.reshape(N, D // 2, 2), jnp.int32)
    seg_i = _sc_segmax()(xi, src, dst)
    seg = lax.bitcast_convert_type(seg_i, jnp.bfloat16).reshape(2 * NPAD, D)
    return _tc_mlp(x, seg[:N], seg[NPAD:NPAD + N], W.T, b.reshape(1, D))
